# Initial kernel scaffold; baseline (speedup 1.0000x reference)
import jax
import jax.numpy as jnp
from jax.experimental import pallas as pl


def kernel(hidden_states, weight, e_score_correction_bias):
    raise NotImplementedError("write your pallas kernel here")


# speedup vs baseline: 1.2301x; 1.2301x over previous
import jax
import jax.numpy as jnp
from jax.experimental import pallas as pl

_TOPK = 8
_NE = 64


def _gate_block(x_ref, w_ref, b_ref, idx_ref, wgt_ref):
    x = x_ref[...]
    w = w_ref[...]
    logits = jax.lax.dot_general(
        x, w, (((1,), (1,)), ((), ())),
        preferred_element_type=jnp.float32)
    logits = logits + b_ref[...]

    bm = logits.shape[0]
    iota = jax.lax.broadcasted_iota(jnp.int32, (bm, _NE), 1)
    vals, idxs = [], []
    l = logits
    for _ in range(_TOPK):
        m = jnp.max(l, axis=1, keepdims=True)
        a = jnp.min(jnp.where(l == m, iota, _NE), axis=1, keepdims=True)
        vals.append(m)
        idxs.append(a)
        l = jnp.where(iota == a, -jnp.inf, l)
    v = jnp.concatenate(vals, axis=1)
    i = jnp.concatenate(idxs, axis=1)
    e = jnp.exp(v - v[:, :1])
    wgt = e / jnp.sum(e, axis=1, keepdims=True)
    idx_ref[...] = i
    wgt_ref[...] = wgt


def kernel(hidden_states, weight, e_score_correction_bias):
    x = hidden_states.reshape(-1, hidden_states.shape[-1])
    m, k = x.shape
    bm = 1024
    b2 = e_score_correction_bias.reshape(1, _NE)
    idx, wgt = pl.pallas_call(
        _gate_block,
        grid=(m // bm,),
        in_specs=[
            pl.BlockSpec((bm, k), lambda i: (i, 0)),
            pl.BlockSpec((_NE, k), lambda i: (0, 0)),
            pl.BlockSpec((1, _NE), lambda i: (0, 0)),
        ],
        out_specs=[
            pl.BlockSpec((bm, _TOPK), lambda i: (i, 0)),
            pl.BlockSpec((bm, _TOPK), lambda i: (i, 0)),
        ],
        out_shape=[
            jax.ShapeDtypeStruct((m, _TOPK), jnp.int32),
            jax.ShapeDtypeStruct((m, _TOPK), jnp.float32),
        ],
    )(x, weight, b2)
    return idx, wgt
